# baseline (speedup 1.0000x reference)
import functools

import jax
import jax.numpy as jnp
from jax.experimental import pallas as pl
from jax.experimental.pallas import tpu as pltpu

_TILE = 2048
_EPAD = 8
_NEG = -1e30


def _gate_kernel(hw, zc_ref, zl_ref, w1c_ref, w1l_ref, b1_ref, w2_ref, b2_ref,
                 oc_ref, ol_ref, okeep_ref, oprobs_ref, ogate_ref, oksum_ref):
    t = pl.program_id(1)
    xc = zc_ref[0]
    xl = zl_ref[0]

    h = (jnp.dot(w1c_ref[...], xc, preferred_element_type=jnp.float32)
         + jnp.dot(w1l_ref[...], xl, preferred_element_type=jnp.float32)
         + b1_ref[...])
    h = h * jax.nn.sigmoid(h)

    logits = jnp.dot(w2_ref[...], h, preferred_element_type=jnp.float32) \
        + b2_ref[...]
    m = jnp.max(logits, axis=0, keepdims=True)
    e = jnp.exp(logits - m)
    p = e / jnp.sum(e, axis=0, keepdims=True)

    amax = jnp.argmax(p, axis=0)
    row = jax.lax.broadcasted_iota(jnp.int32, p.shape, 0)
    g = jnp.where(row == amax[None, :], p, 0.0)

    gc = g[0:1, :]
    gl = g[1:2, :]
    keep = ((gc + gl) > 0).astype(jnp.float32)

    oc_ref[0] = xc * gc
    ol_ref[0] = xl * gl
    okeep_ref[0] = jnp.broadcast_to(keep, p.shape)
    oprobs_ref[0] = p
    ogate_ref[0] = g

    ncols = keep.shape[1]
    col = jax.lax.broadcasted_iota(jnp.int32, (1, ncols), 1) + t * ncols
    s = jnp.sum(jnp.where(col < hw, keep, 0.0))
    blk = jnp.full((1, _EPAD, 128), s, dtype=jnp.float32)

    @pl.when(t == 0)
    def _():
        oksum_ref[...] = blk

    @pl.when(t != 0)
    def _():
        oksum_ref[...] = oksum_ref[...] + blk


@jax.jit
def kernel(z_cam, z_lidar, W1, b1, W2, b2):
    bsz, C, h, w = z_cam.shape
    hw = h * w
    hidden = W1.shape[1]
    E = W2.shape[1]

    zc = z_cam.reshape(bsz, C, hw)
    zl = z_lidar.reshape(bsz, C, hw)
    w1c = W1[:C].T
    w1l = W1[C:].T
    b1c = b1.reshape(hidden, 1)
    w2p = jnp.zeros((_EPAD, hidden), jnp.float32).at[:E].set(W2.T)
    b2p = jnp.full((_EPAD,), _NEG, jnp.float32).at[:E].set(b2).reshape(_EPAD, 1)

    nt = pl.cdiv(hw, _TILE)
    grid = (bsz, nt)

    out_types = (
        jax.ShapeDtypeStruct((bsz, C, hw), jnp.float32),
        jax.ShapeDtypeStruct((bsz, C, hw), jnp.float32),
        jax.ShapeDtypeStruct((bsz, _EPAD, hw), jnp.float32),
        jax.ShapeDtypeStruct((bsz, _EPAD, hw), jnp.float32),
        jax.ShapeDtypeStruct((bsz, _EPAD, hw), jnp.float32),
        jax.ShapeDtypeStruct((bsz, _EPAD, 128), jnp.float32),
    )

    big = pl.BlockSpec((1, C, _TILE), lambda b, t: (b, 0, t))
    small = pl.BlockSpec((1, _EPAD, _TILE), lambda b, t: (b, 0, t))

    oc, ol, okeep, oprobs, ogate, oksum = pl.pallas_call(
        functools.partial(_gate_kernel, hw),
        grid=grid,
        in_specs=[
            big,
            big,
            pl.BlockSpec((hidden, C), lambda b, t: (0, 0)),
            pl.BlockSpec((hidden, C), lambda b, t: (0, 0)),
            pl.BlockSpec((hidden, 1), lambda b, t: (0, 0)),
            pl.BlockSpec((_EPAD, hidden), lambda b, t: (0, 0)),
            pl.BlockSpec((_EPAD, 1), lambda b, t: (0, 0)),
        ],
        out_specs=[
            big, big, small, small, small,
            pl.BlockSpec((1, _EPAD, 128), lambda b, t: (b, 0, 0)),
        ],
        out_shape=out_types,
        compiler_params=pltpu.CompilerParams(
            dimension_semantics=("parallel", "arbitrary"),
        ),
    )(zc, zl, w1c, w1l, b1c, w2p, b2p)

    zhat_cam = oc.reshape(bsz, C, h, w)
    zhat_lidar = ol.reshape(bsz, C, h, w)
    keep_mask_2d = okeep[:, 0:1, :].reshape(bsz, 1, h, w)
    probs = jnp.transpose(oprobs[:, :E, :], (0, 2, 1))
    gate = jnp.transpose(ogate[:, :E, :], (0, 2, 1))
    keep_ratio = oksum[:, 0:1, 0] / jnp.float32(hw)
    return (zhat_cam, zhat_lidar, keep_mask_2d, probs, gate, keep_ratio)


# speedup vs baseline: 1.0221x; 1.0221x over previous
import functools

import jax
import jax.numpy as jnp
from jax.experimental import pallas as pl
from jax.experimental.pallas import tpu as pltpu

_TILE = 4096
_EPAD = 8
_NEG = -1e30


def _gate_kernel(hw, zc_ref, zl_ref, w1c_ref, w1l_ref, b1_ref, w2_ref, b2_ref,
                 oc_ref, ol_ref, okeep_ref, oprobs_ref, ogate_ref, oksum_ref):
    t = pl.program_id(1)
    xc = zc_ref[0]
    xl = zl_ref[0]

    h = (jnp.dot(w1c_ref[...], xc, preferred_element_type=jnp.float32)
         + jnp.dot(w1l_ref[...], xl, preferred_element_type=jnp.float32)
         + b1_ref[...])
    h = h * jax.nn.sigmoid(h)

    logits = jnp.dot(w2_ref[...], h, preferred_element_type=jnp.float32) \
        + b2_ref[...]
    m = jnp.max(logits, axis=0, keepdims=True)
    e = jnp.exp(logits - m)
    p = e / jnp.sum(e, axis=0, keepdims=True)

    amax = jnp.argmax(p, axis=0)
    row = jax.lax.broadcasted_iota(jnp.int32, p.shape, 0)
    g = jnp.where(row == amax[None, :], p, 0.0)

    gc = g[0:1, :]
    gl = g[1:2, :]
    keep = ((gc + gl) > 0).astype(jnp.float32)

    oc_ref[0] = xc * gc
    ol_ref[0] = xl * gl
    okeep_ref[0] = jnp.broadcast_to(keep, p.shape)
    oprobs_ref[0] = p
    ogate_ref[0] = g

    ncols = keep.shape[1]
    col = jax.lax.broadcasted_iota(jnp.int32, (1, ncols), 1) + t * ncols
    s = jnp.sum(jnp.where(col < hw, keep, 0.0))
    blk = jnp.full((1, _EPAD, 128), s, dtype=jnp.float32)

    @pl.when(t == 0)
    def _():
        oksum_ref[...] = blk

    @pl.when(t != 0)
    def _():
        oksum_ref[...] = oksum_ref[...] + blk


@jax.jit
def kernel(z_cam, z_lidar, W1, b1, W2, b2):
    bsz, C, h, w = z_cam.shape
    hw = h * w
    hidden = W1.shape[1]
    E = W2.shape[1]

    zc = z_cam.reshape(bsz, C, hw)
    zl = z_lidar.reshape(bsz, C, hw)
    w1c = W1[:C].T
    w1l = W1[C:].T
    b1c = b1.reshape(hidden, 1)
    w2p = jnp.zeros((_EPAD, hidden), jnp.float32).at[:E].set(W2.T)
    b2p = jnp.full((_EPAD,), _NEG, jnp.float32).at[:E].set(b2).reshape(_EPAD, 1)

    nt = pl.cdiv(hw, _TILE)
    grid = (bsz, nt)

    out_types = (
        jax.ShapeDtypeStruct((bsz, C, hw), jnp.float32),
        jax.ShapeDtypeStruct((bsz, C, hw), jnp.float32),
        jax.ShapeDtypeStruct((bsz, _EPAD, hw), jnp.float32),
        jax.ShapeDtypeStruct((bsz, _EPAD, hw), jnp.float32),
        jax.ShapeDtypeStruct((bsz, _EPAD, hw), jnp.float32),
        jax.ShapeDtypeStruct((bsz, _EPAD, 128), jnp.float32),
    )

    big = pl.BlockSpec((1, C, _TILE), lambda b, t: (b, 0, t))
    small = pl.BlockSpec((1, _EPAD, _TILE), lambda b, t: (b, 0, t))

    oc, ol, okeep, oprobs, ogate, oksum = pl.pallas_call(
        functools.partial(_gate_kernel, hw),
        grid=grid,
        in_specs=[
            big,
            big,
            pl.BlockSpec((hidden, C), lambda b, t: (0, 0)),
            pl.BlockSpec((hidden, C), lambda b, t: (0, 0)),
            pl.BlockSpec((hidden, 1), lambda b, t: (0, 0)),
            pl.BlockSpec((_EPAD, hidden), lambda b, t: (0, 0)),
            pl.BlockSpec((_EPAD, 1), lambda b, t: (0, 0)),
        ],
        out_specs=[
            big, big, small, small, small,
            pl.BlockSpec((1, _EPAD, 128), lambda b, t: (b, 0, 0)),
        ],
        out_shape=out_types,
        compiler_params=pltpu.CompilerParams(
            dimension_semantics=("parallel", "arbitrary"),
        ),
    )(zc, zl, w1c, w1l, b1c, w2p, b2p)

    zhat_cam = oc.reshape(bsz, C, h, w)
    zhat_lidar = ol.reshape(bsz, C, h, w)
    keep_mask_2d = okeep[:, 0:1, :].reshape(bsz, 1, h, w)
    probs = jnp.transpose(oprobs[:, :E, :], (0, 2, 1))
    gate = jnp.transpose(ogate[:, :E, :], (0, 2, 1))
    keep_ratio = oksum[:, 0:1, 0] / jnp.float32(hw)
    return (zhat_cam, zhat_lidar, keep_mask_2d, probs, gate, keep_ratio)
